# baseline (speedup 1.0000x reference)
import jax
import jax.numpy as jnp
from jax import lax
from jax.experimental import pallas as pl
from jax.experimental.pallas import tpu as pltpu
from jax.experimental.pallas import tpu_sc as plsc

B = 4
N = 160000
D = 128
S = 10000
NC = 2
NS = 16

C = 100
CPT = N // (NS * C)
ROWS_PER_TILE = C * CPT
IDROWS = N // C
W = 20
NWIN = CPT // W
SEG_PER_TILE = S // NS


def _seg_body(data_hbm, ids_hbm, out_hbm, idx_w, chunk_a, chunk_b, chunk_c,
              acc_sh, sem_la, sem_lb, sem_lc, sem_i, sem_z):
    c = lax.axis_index("c")
    s = lax.axis_index("s")

    row0 = s * ROWS_PER_TILE
    idrow0 = s * CPT

    def _load(batch, j, buf, sem):
        pltpu.async_copy(
            data_hbm.at[batch, pl.ds(row0 + j * C, C)], buf, sem)

    def _wait_load(batch, buf, sem):
        pltpu.make_async_copy(
            data_hbm.at[batch, pl.ds(row0, C)], buf, sem).wait()

    def _prefire(batch):
        pltpu.async_copy(ids_hbm.at[batch, pl.ds(idrow0, W)], idx_w.at[0],
                         sem_i)
        pltpu.async_copy(ids_hbm.at[batch, pl.ds(idrow0 + W, W)], idx_w.at[1],
                         sem_i)
        _load(batch, 0, chunk_a, sem_la)
        _load(batch, 1, chunk_b, sem_lb)

    def _zfill_c():
        def _zf(k, carry):
            chunk_c[k // (D // 16), pl.ds((k % (D // 16)) * 16, 16)] = (
                jnp.zeros((16,), jnp.float32))
            return carry
        lax.fori_loop(0, C * (D // 16), _zf, 0)

    def _zero_fire_drain():
        zero_d = [
            pltpu.async_copy(
                chunk_c, acc_sh.at[pl.ds(s * SEG_PER_TILE + k * C, C)], sem_z)
            for k in range(SEG_PER_TILE // C)
        ]
        zero_d.append(pltpu.async_copy(
            chunk_c.at[pl.ds(0, SEG_PER_TILE % C)],
            acc_sh.at[pl.ds(s * SEG_PER_TILE + (SEG_PER_TILE // C) * C,
                            SEG_PER_TILE % C)],
            sem_z))
        for d in zero_d:
            d.wait()

    def _main_loop(batch):
        def _step(j, buf, sem, nxt):
            _wait_load(batch, buf, sem)

            @pl.when(j % W == 0)
            def _():
                pltpu.make_async_copy(
                    ids_hbm.at[batch, pl.ds(idrow0, W)], idx_w.at[0],
                    sem_i).wait()

            if nxt is not None:
                nb, nsem = nxt
                _load(batch, j + 2, nb, nsem)

            w = j // W
            pltpu.sync_copy(buf, acc_sh.at[idx_w.at[w % 2, j % W]], add=True)

            @pl.when((j % W == W - 1) & (w + 2 < NWIN))
            def _():
                pltpu.async_copy(
                    ids_hbm.at[batch, pl.ds(idrow0 + (w + 2) * W, W)],
                    idx_w.at[w % 2], sem_i)

        bufs = {0: (chunk_a, sem_la), 1: (chunk_b, sem_lb),
                2: (chunk_c, sem_lc)}

        _step(0, chunk_a, sem_la, bufs[2])
        _step(1, chunk_b, sem_lb, bufs[0])

        def _tri(t, carry):
            j = 3 * t + 2
            _step(j, chunk_c, sem_lc, bufs[1])
            _step(j + 1, chunk_a, sem_la, bufs[2])
            _step(j + 2, chunk_b, sem_lb, bufs[0])
            return carry
        lax.fori_loop(0, (CPT - 7) // 3, _tri, 0)

        for j in range(CPT - 5, CPT):
            _step(j, *bufs[j % 3],
                  bufs[(j + 2) % 3] if j + 2 < CPT else None)

    def _copy_out(batch):
        pltpu.sync_copy(
            acc_sh.at[pl.ds(s * SEG_PER_TILE, SEG_PER_TILE)],
            out_hbm.at[batch, pl.ds(s * SEG_PER_TILE, SEG_PER_TILE)])

    b0 = c * (B // NC)
    _prefire(b0)
    _zfill_c()
    _zero_fire_drain()
    plsc.subcore_barrier()
    _main_loop(b0)
    _zfill_c()
    plsc.subcore_barrier()

    _prefire(b0 + 1)
    _copy_out(b0)
    _zero_fire_drain()
    plsc.subcore_barrier()
    _main_loop(b0 + 1)
    plsc.subcore_barrier()
    _copy_out(b0 + 1)


@jax.jit
def kernel(data, segment_ids):
    ids32 = segment_ids.astype(jnp.int32).reshape(B, IDROWS, C)
    mesh = plsc.VectorSubcoreMesh(core_axis_name="c", subcore_axis_name="s")
    return pl.kernel(
        _seg_body,
        out_type=jax.ShapeDtypeStruct((B, S, D), jnp.float32),
        mesh=mesh,
        compiler_params=pltpu.CompilerParams(use_tc_tiling_on_sc=False),
        scratch_types=[
            pltpu.VMEM((2, W, C), jnp.int32),
            pltpu.VMEM((C, D), jnp.float32),
            pltpu.VMEM((C, D), jnp.float32),
            pltpu.VMEM((C, D), jnp.float32),
            pltpu.VMEM_SHARED((S, D), jnp.float32),
            pltpu.SemaphoreType.DMA,
            pltpu.SemaphoreType.DMA,
            pltpu.SemaphoreType.DMA,
            pltpu.SemaphoreType.DMA,
            pltpu.SemaphoreType.DMA,
        ],
    )(data, ids32)


# speedup vs baseline: 1.0027x; 1.0027x over previous
import jax
import jax.numpy as jnp
from jax import lax
from jax.experimental import pallas as pl
from jax.experimental.pallas import tpu as pltpu
from jax.experimental.pallas import tpu_sc as plsc

B = 4
N = 160000
D = 128
S = 10000
NC = 2
NS = 16

C = 100
CPT = N // (NS * C)
ROWS_PER_TILE = C * CPT
IDROWS = N // C
SEG_PER_TILE = S // NS


def _seg_body(data_hbm, ids_hbm, out_hbm, idx_w, chunk_a, chunk_b, chunk_c,
              acc_sh, sem_la, sem_lb, sem_lc, sem_i, sem_z):
    c = lax.axis_index("c")
    s = lax.axis_index("s")

    row0 = s * ROWS_PER_TILE
    idrow0 = s * CPT

    def _load(batch, j, buf, sem):
        pltpu.async_copy(
            data_hbm.at[batch, pl.ds(row0 + j * C, C)], buf, sem)

    def _wait_load(batch, buf, sem):
        pltpu.make_async_copy(
            data_hbm.at[batch, pl.ds(row0, C)], buf, sem).wait()

    def _prefire(batch):
        pltpu.async_copy(ids_hbm.at[batch, pl.ds(idrow0, CPT)], idx_w, sem_i)
        _load(batch, 0, chunk_a, sem_la)
        _load(batch, 1, chunk_b, sem_lb)

    def _zfill_c():
        def _zf(k, carry):
            chunk_c[k // (D // 16), pl.ds((k % (D // 16)) * 16, 16)] = (
                jnp.zeros((16,), jnp.float32))
            return carry
        lax.fori_loop(0, C * (D // 16), _zf, 0)

    def _zero_fire_drain():
        zero_d = [
            pltpu.async_copy(
                chunk_c, acc_sh.at[pl.ds(s * SEG_PER_TILE + k * C, C)], sem_z)
            for k in range(SEG_PER_TILE // C)
        ]
        zero_d.append(pltpu.async_copy(
            chunk_c.at[pl.ds(0, SEG_PER_TILE % C)],
            acc_sh.at[pl.ds(s * SEG_PER_TILE + (SEG_PER_TILE // C) * C,
                            SEG_PER_TILE % C)],
            sem_z))
        for d in zero_d:
            d.wait()

    def _ids_drain(batch):
        pltpu.make_async_copy(
            ids_hbm.at[batch, pl.ds(idrow0, CPT)], idx_w, sem_i).wait()

    def _main_loop(batch):
        def _step(j, buf, sem, nxt):
            _wait_load(batch, buf, sem)
            if nxt is not None:
                nb, nsem = nxt
                _load(batch, j + 2, nb, nsem)
            pltpu.sync_copy(buf, acc_sh.at[idx_w.at[j]], add=True)

        bufs = {0: (chunk_a, sem_la), 1: (chunk_b, sem_lb),
                2: (chunk_c, sem_lc)}

        _step(0, chunk_a, sem_la, bufs[2])
        _step(1, chunk_b, sem_lb, bufs[0])

        def _tri(t, carry):
            j = 3 * t + 2
            _step(j, chunk_c, sem_lc, bufs[1])
            _step(j + 1, chunk_a, sem_la, bufs[2])
            _step(j + 2, chunk_b, sem_lb, bufs[0])
            return carry
        lax.fori_loop(0, (CPT - 7) // 3, _tri, 0)

        for j in range(CPT - 5, CPT):
            _step(j, *bufs[j % 3],
                  bufs[(j + 2) % 3] if j + 2 < CPT else None)

    def _copy_out(batch):
        pltpu.sync_copy(
            acc_sh.at[pl.ds(s * SEG_PER_TILE, SEG_PER_TILE)],
            out_hbm.at[batch, pl.ds(s * SEG_PER_TILE, SEG_PER_TILE)])

    b0 = c * (B // NC)
    _prefire(b0)
    _zfill_c()
    _zero_fire_drain()
    _ids_drain(b0)
    plsc.subcore_barrier()
    _main_loop(b0)
    _zfill_c()
    plsc.subcore_barrier()

    _prefire(b0 + 1)
    _copy_out(b0)
    _zero_fire_drain()
    _ids_drain(b0 + 1)
    plsc.subcore_barrier()
    _main_loop(b0 + 1)
    plsc.subcore_barrier()
    _copy_out(b0 + 1)


@jax.jit
def kernel(data, segment_ids):
    ids32 = segment_ids.astype(jnp.int32).reshape(B, IDROWS, C)
    mesh = plsc.VectorSubcoreMesh(core_axis_name="c", subcore_axis_name="s")
    return pl.kernel(
        _seg_body,
        out_type=jax.ShapeDtypeStruct((B, S, D), jnp.float32),
        mesh=mesh,
        compiler_params=pltpu.CompilerParams(use_tc_tiling_on_sc=False),
        scratch_types=[
            pltpu.VMEM((CPT, C), jnp.int32),
            pltpu.VMEM((C, D), jnp.float32),
            pltpu.VMEM((C, D), jnp.float32),
            pltpu.VMEM((C, D), jnp.float32),
            pltpu.VMEM_SHARED((S, D), jnp.float32),
            pltpu.SemaphoreType.DMA,
            pltpu.SemaphoreType.DMA,
            pltpu.SemaphoreType.DMA,
            pltpu.SemaphoreType.DMA,
            pltpu.SemaphoreType.DMA,
        ],
    )(data, ids32)


# speedup vs baseline: 1.0599x; 1.0570x over previous
import jax
import jax.numpy as jnp
from jax import lax
from jax.experimental import pallas as pl
from jax.experimental.pallas import tpu as pltpu
from jax.experimental.pallas import tpu_sc as plsc

B = 4
N = 160000
D = 128
S = 10000
NC = 2
NS = 16

C = 80
CPT = N // (NS * C)
TAIL = ((CPT - 4) % 3) + 2
ROWS_PER_TILE = C * CPT
IDROWS = N // C
SEG_PER_TILE = S // NS


def _seg_body(data_hbm, ids_hbm, out_hbm, idx_w, chunk_a, chunk_b, chunk_c,
              acc_sh, sem_la, sem_lb, sem_lc, sem_i, sem_z):
    c = lax.axis_index("c")
    s = lax.axis_index("s")

    row0 = s * ROWS_PER_TILE
    idrow0 = s * CPT

    def _load(batch, j, buf, sem):
        pltpu.async_copy(
            data_hbm.at[batch, pl.ds(row0 + j * C, C)], buf, sem)

    def _wait_load(batch, buf, sem):
        pltpu.make_async_copy(
            data_hbm.at[batch, pl.ds(row0, C)], buf, sem).wait()

    def _prefire(batch):
        pltpu.async_copy(ids_hbm.at[batch, pl.ds(idrow0, CPT)], idx_w, sem_i)
        _load(batch, 0, chunk_a, sem_la)
        _load(batch, 1, chunk_b, sem_lb)

    def _zfill_c():
        def _zf(k, carry):
            chunk_c[k // (D // 16), pl.ds((k % (D // 16)) * 16, 16)] = (
                jnp.zeros((16,), jnp.float32))
            return carry
        lax.fori_loop(0, C * (D // 16), _zf, 0)

    def _zero_fire_drain():
        zero_d = [
            pltpu.async_copy(
                chunk_c, acc_sh.at[pl.ds(s * SEG_PER_TILE + k * C, C)], sem_z)
            for k in range(SEG_PER_TILE // C)
        ]
        zero_d.append(pltpu.async_copy(
            chunk_c.at[pl.ds(0, SEG_PER_TILE % C)],
            acc_sh.at[pl.ds(s * SEG_PER_TILE + (SEG_PER_TILE // C) * C,
                            SEG_PER_TILE % C)],
            sem_z))
        for d in zero_d:
            d.wait()

    def _ids_drain(batch):
        pltpu.make_async_copy(
            ids_hbm.at[batch, pl.ds(idrow0, CPT)], idx_w, sem_i).wait()

    def _main_loop(batch):
        def _step(j, buf, sem, nxt):
            _wait_load(batch, buf, sem)
            if nxt is not None:
                nb, nsem = nxt
                _load(batch, j + 2, nb, nsem)
            pltpu.sync_copy(buf, acc_sh.at[idx_w.at[j]], add=True)

        bufs = {0: (chunk_a, sem_la), 1: (chunk_b, sem_lb),
                2: (chunk_c, sem_lc)}

        _step(0, chunk_a, sem_la, bufs[2])
        _step(1, chunk_b, sem_lb, bufs[0])

        def _tri(t, carry):
            j = 3 * t + 2
            _step(j, chunk_c, sem_lc, bufs[1])
            _step(j + 1, chunk_a, sem_la, bufs[2])
            _step(j + 2, chunk_b, sem_lb, bufs[0])
            return carry
        lax.fori_loop(0, (CPT - 2 - TAIL) // 3, _tri, 0)

        for j in range(CPT - TAIL, CPT):
            _step(j, *bufs[j % 3],
                  bufs[(j + 2) % 3] if j + 2 < CPT else None)

    def _copy_out(batch):
        pltpu.sync_copy(
            acc_sh.at[pl.ds(s * SEG_PER_TILE, SEG_PER_TILE)],
            out_hbm.at[batch, pl.ds(s * SEG_PER_TILE, SEG_PER_TILE)])

    b0 = c * (B // NC)
    _prefire(b0)
    _zfill_c()
    _zero_fire_drain()
    _ids_drain(b0)
    plsc.subcore_barrier()
    _main_loop(b0)
    _zfill_c()
    plsc.subcore_barrier()

    _prefire(b0 + 1)
    _copy_out(b0)
    _zero_fire_drain()
    _ids_drain(b0 + 1)
    plsc.subcore_barrier()
    _main_loop(b0 + 1)
    plsc.subcore_barrier()
    _copy_out(b0 + 1)


@jax.jit
def kernel(data, segment_ids):
    ids32 = segment_ids.astype(jnp.int32).reshape(B, IDROWS, C)
    mesh = plsc.VectorSubcoreMesh(core_axis_name="c", subcore_axis_name="s")
    return pl.kernel(
        _seg_body,
        out_type=jax.ShapeDtypeStruct((B, S, D), jnp.float32),
        mesh=mesh,
        compiler_params=pltpu.CompilerParams(use_tc_tiling_on_sc=False),
        scratch_types=[
            pltpu.VMEM((CPT, C), jnp.int32),
            pltpu.VMEM((C, D), jnp.float32),
            pltpu.VMEM((C, D), jnp.float32),
            pltpu.VMEM((C, D), jnp.float32),
            pltpu.VMEM_SHARED((S, D), jnp.float32),
            pltpu.SemaphoreType.DMA,
            pltpu.SemaphoreType.DMA,
            pltpu.SemaphoreType.DMA,
            pltpu.SemaphoreType.DMA,
            pltpu.SemaphoreType.DMA,
        ],
    )(data, ids32)


# speedup vs baseline: 1.0802x; 1.0192x over previous
import jax
import jax.numpy as jnp
from jax import lax
from jax.experimental import pallas as pl
from jax.experimental.pallas import tpu as pltpu
from jax.experimental.pallas import tpu_sc as plsc

B = 4
N = 160000
D = 128
S = 10000
NC = 2
NS = 16

C = 80
CPT = N // (NS * C)
TAIL = ((CPT - 4) % 3) + 2
ROWS_PER_TILE = C * CPT
IDROWS = N // C
SEG_PER_TILE = S // NS
ZROWS = 25


def _seg_body(data_hbm, ids_hbm, out_hbm, idx_w, chunk_a, chunk_b, chunk_c,
              zero_v, acc_sh, sem_la, sem_lb, sem_lc, sem_i, sem_z):
    c = lax.axis_index("c")
    s = lax.axis_index("s")

    def _zf(k, carry):
        zero_v[k // (D // 16), pl.ds((k % (D // 16)) * 16, 16)] = (
            jnp.zeros((16,), jnp.float32))
        return carry
    lax.fori_loop(0, ZROWS * (D // 16), _zf, 0)

    row0 = s * ROWS_PER_TILE
    idrow0 = s * CPT

    def _load(batch, j, buf, sem):
        pltpu.async_copy(
            data_hbm.at[batch, pl.ds(row0 + j * C, C)], buf, sem)

    def _wait_load(batch, buf, sem):
        pltpu.make_async_copy(
            data_hbm.at[batch, pl.ds(row0, C)], buf, sem).wait()

    def _prefire(batch):
        pltpu.async_copy(ids_hbm.at[batch, pl.ds(idrow0, CPT)], idx_w, sem_i)
        _load(batch, 0, chunk_a, sem_la)
        _load(batch, 1, chunk_b, sem_lb)

    def _zero_fire_drain():
        zero_d = [
            pltpu.async_copy(
                zero_v, acc_sh.at[pl.ds(s * SEG_PER_TILE + k * ZROWS, ZROWS)],
                sem_z)
            for k in range(SEG_PER_TILE // ZROWS)
        ]
        for d in zero_d:
            d.wait()

    def _ids_drain(batch):
        pltpu.make_async_copy(
            ids_hbm.at[batch, pl.ds(idrow0, CPT)], idx_w, sem_i).wait()

    def _main_loop(batch):
        def _step(j, buf, sem, nxt):
            _wait_load(batch, buf, sem)
            if nxt is not None:
                nb, nsem = nxt
                _load(batch, j + 2, nb, nsem)
            pltpu.sync_copy(buf, acc_sh.at[idx_w.at[j]], add=True)

        bufs = {0: (chunk_a, sem_la), 1: (chunk_b, sem_lb),
                2: (chunk_c, sem_lc)}

        _step(0, chunk_a, sem_la, bufs[2])
        _step(1, chunk_b, sem_lb, bufs[0])

        def _tri(t, carry):
            j = 3 * t + 2
            _step(j, chunk_c, sem_lc, bufs[1])
            _step(j + 1, chunk_a, sem_la, bufs[2])
            _step(j + 2, chunk_b, sem_lb, bufs[0])
            return carry
        lax.fori_loop(0, (CPT - 2 - TAIL) // 3, _tri, 0)

        for j in range(CPT - TAIL, CPT):
            _step(j, *bufs[j % 3],
                  bufs[(j + 2) % 3] if j + 2 < CPT else None)

    def _copy_out(batch):
        pltpu.sync_copy(
            acc_sh.at[pl.ds(s * SEG_PER_TILE, SEG_PER_TILE)],
            out_hbm.at[batch, pl.ds(s * SEG_PER_TILE, SEG_PER_TILE)])

    b0 = c * (B // NC)
    _prefire(b0)
    _zero_fire_drain()
    _ids_drain(b0)
    plsc.subcore_barrier()
    _main_loop(b0)
    plsc.subcore_barrier()

    _prefire(b0 + 1)
    _copy_out(b0)
    _zero_fire_drain()
    _ids_drain(b0 + 1)
    plsc.subcore_barrier()
    _main_loop(b0 + 1)
    plsc.subcore_barrier()
    _copy_out(b0 + 1)


@jax.jit
def kernel(data, segment_ids):
    ids32 = segment_ids.astype(jnp.int32).reshape(B, IDROWS, C)
    mesh = plsc.VectorSubcoreMesh(core_axis_name="c", subcore_axis_name="s")
    return pl.kernel(
        _seg_body,
        out_type=jax.ShapeDtypeStruct((B, S, D), jnp.float32),
        mesh=mesh,
        compiler_params=pltpu.CompilerParams(use_tc_tiling_on_sc=False),
        scratch_types=[
            pltpu.VMEM((CPT, C), jnp.int32),
            pltpu.VMEM((C, D), jnp.float32),
            pltpu.VMEM((C, D), jnp.float32),
            pltpu.VMEM((C, D), jnp.float32),
            pltpu.VMEM((ZROWS, D), jnp.float32),
            pltpu.VMEM_SHARED((S, D), jnp.float32),
            pltpu.SemaphoreType.DMA,
            pltpu.SemaphoreType.DMA,
            pltpu.SemaphoreType.DMA,
            pltpu.SemaphoreType.DMA,
            pltpu.SemaphoreType.DMA,
        ],
    )(data, ids32)
